# baseline (speedup 1.0000x reference)
import jax
import jax.numpy as jnp
from jax import lax
from jax.experimental import pallas as pl
from jax.experimental.pallas import tpu as pltpu
from jax.experimental.pallas import tpu_sc as plsc

N_CLS = 1000
ROWS = 4096 * 20
NC, NS, L = 2, 16, 16
NW = NC * NS
R_PER_W = ROWS // NW
CHUNK = 32
N_CHUNKS = R_PER_W // CHUNK
NBUF = 2


def _sc_onehot(x_hbm, out_hbm, idx_v, buf0, buf1, sem0, sem1):
    wid = lax.axis_index("s") * NC + lax.axis_index("c")
    row0 = wid * R_PER_W
    sems = (sem0, sem1)
    bufs = (buf0, buf1)

    pltpu.sync_copy(x_hbm.at[pl.ds(row0, R_PER_W)], idx_v)

    for b in range(NBUF):
        buf = bufs[b]

        def _zero(i, _, buf=buf):
            buf[pl.ds(i * L, L)] = jnp.zeros((L,), jnp.float32)
            return 0
        lax.fori_loop(0, (CHUNK * N_CLS) // L, _zero, 0)

    lane = lax.iota(jnp.int32, L) * N_CLS
    ones = jnp.full((L,), 1.0, jnp.float32)
    zeros = jnp.zeros((L,), jnp.float32)

    def _flats(g):
        out = []
        for j in range(CHUNK // L):
            cols = idx_v[pl.ds(g * CHUNK + j * L, L)]
            out.append(lane + (j * L * N_CLS) + cols)
        return out

    def _dst(g):
        return out_hbm.at[pl.ds((row0 + g * CHUNK) * N_CLS, CHUNK * N_CLS)]

    def _outer(o, _):
        for b in range(NBUF):
            g = o * NBUF + b
            buf = bufs[b]

            @pl.when(o > 0)
            def _drain():
                pltpu.make_async_copy(buf, _dst(g - NBUF), sems[b]).wait()
                for f in _flats(g - NBUF):
                    plsc.store_scatter(buf, [f], zeros)

            for f in _flats(g):
                plsc.store_scatter(buf, [f], ones)
            pltpu.async_copy(buf, _dst(g), sems[b])
        return 0

    lax.fori_loop(0, N_CHUNKS // NBUF, _outer, 0)

    for b in range(NBUF):
        g_last = N_CHUNKS - NBUF + b
        pltpu.make_async_copy(bufs[b], _dst(g_last), sems[b]).wait()


def kernel(x):
    xf = x.reshape(ROWS).astype(jnp.int32)
    mesh = plsc.VectorSubcoreMesh(core_axis_name="c", subcore_axis_name="s")
    out = pl.kernel(
        _sc_onehot,
        out_type=jax.ShapeDtypeStruct((ROWS * N_CLS,), jnp.float32),
        mesh=mesh,
        scratch_types=[
            pltpu.VMEM((R_PER_W,), jnp.int32),
            pltpu.VMEM((CHUNK * N_CLS,), jnp.float32),
            pltpu.VMEM((CHUNK * N_CLS,), jnp.float32),
            pltpu.SemaphoreType.DMA,
            pltpu.SemaphoreType.DMA,
        ],
        compiler_params=pltpu.CompilerParams(needs_layout_passes=False),
    )(xf)
    return out.reshape(4096, 20, N_CLS)


# speedup vs baseline: 1.4757x; 1.4757x over previous
import jax
import jax.numpy as jnp
from jax import lax
from jax.experimental import pallas as pl
from jax.experimental.pallas import tpu as pltpu
from jax.experimental.pallas import tpu_sc as plsc

N_CLS = 1000
D0, D1 = 4096, 20
ROWS = D0 * D1
NC, NS, L = 2, 16, 16
NW = NC * NS
R_PER_W = ROWS // NW
D0_PER_W = D0 // NW
CB = 4
FLAT_PER_CHUNK = CB * D1
N_CHUNKS = D0_PER_W // CB


def _sc_onehot(x_hbm, z_hbm, out_hbm, idx_v, buf):
    wid = lax.axis_index("s") * NC + lax.axis_index("c")

    pltpu.sync_copy(x_hbm.at[pl.ds(wid * R_PER_W, R_PER_W)], idx_v)
    pltpu.sync_copy(z_hbm, buf)

    iota = lax.iota(jnp.int32, L)
    ones = jnp.full((L,), 1.0, jnp.float32)
    zeros = jnp.zeros((L,), jnp.float32)
    coords = []
    for j in range(FLAT_PER_CHUNK // L):
        fl = iota + (j * L)
        coords.append((fl // D1, fl % D1))

    def _chunk(g, _):
        flats = []
        for j, (d0l, d1l) in enumerate(coords):
            cols = idx_v[pl.ds(g * FLAT_PER_CHUNK + j * L, L)]
            flats.append((d0l, d1l, cols))
        for d0l, d1l, cols in flats:
            plsc.store_scatter(buf, [d0l, d1l, cols], ones)
        pltpu.sync_copy(buf, out_hbm.at[pl.ds(wid * D0_PER_W + g * CB, CB)])
        for d0l, d1l, cols in flats:
            plsc.store_scatter(buf, [d0l, d1l, cols], zeros)
        return 0

    lax.fori_loop(0, N_CHUNKS, _chunk, 0)


def kernel(x):
    xf = x.reshape(ROWS).astype(jnp.int32)
    zblk = jnp.zeros((CB, D1, N_CLS), jnp.float32)
    mesh = plsc.VectorSubcoreMesh(core_axis_name="c", subcore_axis_name="s")
    out = pl.kernel(
        _sc_onehot,
        out_type=jax.ShapeDtypeStruct((D0, D1, N_CLS), jnp.float32),
        mesh=mesh,
        scratch_types=[
            pltpu.VMEM((R_PER_W,), jnp.int32),
            pltpu.VMEM((CB, D1, N_CLS), jnp.float32),
        ],
        compiler_params=pltpu.CompilerParams(needs_layout_passes=False),
    )(xf, zblk)
    return out


# speedup vs baseline: 5.2767x; 3.5757x over previous
import jax
import jax.numpy as jnp
from jax import lax
from jax.experimental import pallas as pl
from jax.experimental.pallas import tpu as pltpu
from jax.experimental.pallas import tpu_sc as plsc

N_CLS = 1000
D0, D1 = 4096, 20
NC, NS, L = 2, 16, 16
NW = NC * NS
SLAB = D0 // NW


def _sc_onehot(xt_hbm, z_hbm, out_hbm, xv, buf):
    wid = lax.axis_index("s") * NC + lax.axis_index("c")
    d0_0 = wid * SLAB

    pltpu.sync_copy(xt_hbm.at[:, pl.ds(d0_0, SLAB)], xv)
    pltpu.sync_copy(z_hbm, buf)

    iota = lax.iota(jnp.int32, L)
    zero16 = jnp.zeros((L,), jnp.int32)
    ones = jnp.full((L,), 1.0, jnp.float32)
    zeros = jnp.zeros((L,), jnp.float32)

    def _chunk(d1, _):
        flats = []
        for j in range(SLAB // L):
            cols = xv[d1, pl.ds(j * L, L)]
            flats.append((cols, iota + j * L))
        for cols, d0l in flats:
            plsc.store_scatter(buf, [zero16, cols, d0l], ones)
        pltpu.sync_copy(
            buf, out_hbm.at[pl.ds(d1, 1), :, pl.ds(d0_0, SLAB)]
        )
        for cols, d0l in flats:
            plsc.store_scatter(buf, [zero16, cols, d0l], zeros)
        return 0

    lax.fori_loop(0, D1, _chunk, 0)


def kernel(x):
    xt = jnp.transpose(x.astype(jnp.int32))
    zblk = jnp.zeros((1, N_CLS, SLAB), jnp.float32)
    mesh = plsc.VectorSubcoreMesh(core_axis_name="c", subcore_axis_name="s")
    out = pl.kernel(
        _sc_onehot,
        out_type=jax.ShapeDtypeStruct((D1, N_CLS, D0), jnp.float32),
        mesh=mesh,
        scratch_types=[
            pltpu.VMEM((D1, SLAB), jnp.int32),
            pltpu.VMEM((1, N_CLS, SLAB), jnp.float32),
        ],
        compiler_params=pltpu.CompilerParams(needs_layout_passes=False),
    )(xt, zblk)
    return jnp.transpose(out, (2, 0, 1))
